# baseline (speedup 1.0000x reference)
import dataclasses
import functools

import jax
import jax.numpy as jnp
from jax import lax
from jax.experimental import pallas as pl
from jax.experimental.pallas import tpu as pltpu
from jax.experimental.pallas import tpu_sc as plsc

N_NODES = 10000
N_EDGES = 320000
IN_F = 128
OUT_F = 128

NC = 2
NS = 16
NW = NC * NS
L = 16
CB = 64
KQ = 40
KC0 = 240
KC1 = 80
DEPTH = 4
TOT_CHUNKS = NS * (KC0 + KC1)
E_PAD = TOT_CHUNKS * CB
N_PAD = 10240
RPS = N_PAD // NS

_f32 = jnp.float32


def _mm_body(x_ref, w_ref, o_ref):
    o_ref[...] = jnp.dot(x_ref[...], w_ref[...],
                         preferred_element_type=_f32)


def _matmul(x, W):
    grid = 10
    blk = N_NODES // grid
    return pl.pallas_call(
        _mm_body,
        grid=(grid,),
        in_specs=[
            pl.BlockSpec((blk, IN_F), lambda i: (i, 0)),
            pl.BlockSpec((IN_F, OUT_F), lambda i: (0, 0)),
        ],
        out_specs=pl.BlockSpec((blk, OUT_F), lambda i: (i, 0)),
        out_shape=jax.ShapeDtypeStruct((N_NODES, OUT_F), _f32),
    )(x, W)


def _sc_body(h_hbm, src_hbm, dst_hbm, w_hbm, out_hbm,
             acc, src_v, dst_v, w_v, *rest):
    bufs = rest[:DEPTH]
    sems = rest[DEPTH:]
    c = lax.axis_index("c")
    s = lax.axis_index("s")

    z = bufs[0]
    @pl.loop(0, CB)
    def _(i):
        for j in range(OUT_F // L):
            z[i, pl.ds(j * L, L)] = jnp.zeros((L,), _f32)

    base = s * RPS
    for t in range(RPS // CB):
        pltpu.sync_copy(z, acc.at[pl.ds(base + t * CB, CB)])
    plsc.subcore_barrier()

    def pipeline(chunk0, nph):
        @pl.loop(0, nph)
        def _(p):
            off = pl.multiple_of(chunk0 + p * KQ, 8)
            pltpu.sync_copy(src_hbm.at[pl.ds(off, KQ)], src_v)
            pltpu.sync_copy(dst_hbm.at[pl.ds(off, KQ)], dst_v)
            pltpu.sync_copy(w_hbm.at[pl.ds(off, KQ)], w_v)

            for b in range(DEPTH):
                pltpu.async_copy(h_hbm.at[src_v.at[b]], bufs[b], sems[b])

            @pl.loop(0, KQ, step=DEPTH)
            def _(k):
                for b in range(DEPTH):
                    cur, csem = bufs[b], sems[b]
                    kk = k + b
                    pltpu.make_async_copy(h_hbm.at[src_v.at[kk]], cur,
                                          csem).wait()

                    @plsc.parallel_loop(0, CB, unroll=4)
                    def _(i):
                        wv = plsc.load_gather(
                            w_v.at[kk], [jnp.full((L,), i, jnp.int32)])
                        for j in range(OUT_F // L):
                            sl = pl.ds(j * L, L)
                            cur[i, sl] = cur[i, sl] * wv

                    pltpu.sync_copy(cur, acc.at[dst_v.at[kk]], add=True)

                    @pl.when(kk + DEPTH < KQ)
                    def _():
                        pltpu.async_copy(h_hbm.at[src_v.at[kk + DEPTH]],
                                         cur, csem)

    @pl.when(c == 0)
    def _():
        pipeline(s * KC0, KC0 // KQ)

    @pl.when(c == 1)
    def _():
        pipeline(NS * KC0 + s * KC1, KC1 // KQ)

    plsc.subcore_barrier()
    pltpu.sync_copy(acc.at[pl.ds(base, RPS)],
                    out_hbm.at[c, pl.ds(base, RPS)])


def _sc_scatter(h, src3, dst3, w3):
    mesh = plsc.VectorSubcoreMesh(core_axis_name="c", subcore_axis_name="s")
    cp = pltpu.CompilerParams()
    if "needs_layout_passes" in pltpu.CompilerParams.__dataclass_fields__:
        cp = dataclasses.replace(cp, needs_layout_passes=False)
    fn = pl.kernel(
        _sc_body,
        mesh=mesh,
        out_type=jax.ShapeDtypeStruct((NC, N_PAD, OUT_F), _f32),
        scratch_types=[
            pltpu.VMEM_SHARED((N_PAD, OUT_F), _f32),
            pltpu.VMEM((KQ, CB), jnp.int32),
            pltpu.VMEM((KQ, CB), jnp.int32),
            pltpu.VMEM((KQ, CB), _f32),
        ] + [pltpu.VMEM((CB, OUT_F), _f32) for _ in range(DEPTH)]
          + [pltpu.SemaphoreType.DMA for _ in range(DEPTH)],
        compiler_params=cp,
    )
    return fn(h, src3, dst3, w3)


def _combine_body(acc_ref, b_ref, o_ref):
    o_ref[...] = acc_ref[0] + acc_ref[1] + b_ref[...]


def _combine(acc, b):
    grid = 10
    blk = N_NODES // grid
    return pl.pallas_call(
        _combine_body,
        grid=(grid,),
        in_specs=[
            pl.BlockSpec((NC, blk, OUT_F), lambda i: (0, i, 0)),
            pl.BlockSpec((1, OUT_F), lambda i: (0, 0)),
        ],
        out_specs=pl.BlockSpec((blk, OUT_F), lambda i: (i, 0)),
        out_shape=jax.ShapeDtypeStruct((N_NODES, OUT_F), _f32),
    )(acc, b.reshape(1, OUT_F))


def kernel(x, edge_index, edge_weight, W, b):
    src = edge_index[1].astype(jnp.int32)
    dst = edge_index[0].astype(jnp.int32)
    pad = E_PAD - N_EDGES
    src3 = jnp.pad(src, (0, pad)).reshape(TOT_CHUNKS, CB)
    dst3 = jnp.pad(dst, (0, pad)).reshape(TOT_CHUNKS, CB)
    w3 = jnp.pad(edge_weight.astype(_f32), (0, pad)).reshape(TOT_CHUNKS, CB)

    h = _matmul(x.astype(_f32), W.astype(_f32))
    acc = _sc_scatter(h, src3, dst3, w3)
    return _combine(acc, b.astype(_f32))


# speedup vs baseline: 1.3162x; 1.3162x over previous
import dataclasses
import functools

import jax
import jax.numpy as jnp
from jax import lax
from jax.experimental import pallas as pl
from jax.experimental.pallas import tpu as pltpu
from jax.experimental.pallas import tpu_sc as plsc

N_NODES = 10000
N_EDGES = 320000
IN_F = 128
OUT_F = 128

NC = 2
NS = 16
NW = NC * NS
L = 16
CB = 128
KQ = 16
KC0 = 120
KC1 = 40
DEPTH = 2
TOT_CHUNKS = NS * (KC0 + KC1)
E_PAD = TOT_CHUNKS * CB
N_PAD = 10240
RPS = N_PAD // NS

_f32 = jnp.float32


def _mm_body(x_ref, w_ref, o_ref):
    o_ref[...] = jnp.dot(x_ref[...], w_ref[...],
                         preferred_element_type=_f32)


def _matmul(x, W):
    grid = 10
    blk = N_NODES // grid
    return pl.pallas_call(
        _mm_body,
        grid=(grid,),
        in_specs=[
            pl.BlockSpec((blk, IN_F), lambda i: (i, 0)),
            pl.BlockSpec((IN_F, OUT_F), lambda i: (0, 0)),
        ],
        out_specs=pl.BlockSpec((blk, OUT_F), lambda i: (i, 0)),
        out_shape=jax.ShapeDtypeStruct((N_NODES, OUT_F), _f32),
    )(x, W)


def _sc_body(h_hbm, src_hbm, dst_hbm, w_hbm, out_hbm,
             acc, src_v, dst_v, w_v, *rest):
    bufs = rest[:DEPTH]
    sems = rest[DEPTH:]
    c = lax.axis_index("c")
    s = lax.axis_index("s")

    z = bufs[0]
    @pl.loop(0, CB)
    def _(i):
        for j in range(OUT_F // L):
            z[i, pl.ds(j * L, L)] = jnp.zeros((L,), _f32)

    base = s * RPS
    for t in range(RPS // CB):
        pltpu.sync_copy(z, acc.at[pl.ds(base + t * CB, CB)])
    plsc.subcore_barrier()

    def pipeline(chunk0, nph):
        @pl.loop(0, nph)
        def _(p):
            off = pl.multiple_of(chunk0 + p * KQ, 8)
            pltpu.sync_copy(src_hbm.at[pl.ds(off, KQ)], src_v)
            pltpu.sync_copy(dst_hbm.at[pl.ds(off, KQ)], dst_v)
            pltpu.sync_copy(w_hbm.at[pl.ds(off, KQ)], w_v)

            for b in range(DEPTH):
                pltpu.async_copy(h_hbm.at[src_v.at[b]], bufs[b], sems[b])

            @pl.loop(0, KQ, step=DEPTH)
            def _(k):
                for b in range(DEPTH):
                    cur, csem = bufs[b], sems[b]
                    kk = k + b
                    pltpu.make_async_copy(h_hbm.at[src_v.at[kk]], cur,
                                          csem).wait()

                    @pl.loop(0, CB)
                    def _(i):
                        wv = plsc.load_gather(
                            w_v.at[kk], [jnp.full((L,), i, jnp.int32)])
                        for j in range(OUT_F // L):
                            sl = pl.ds(j * L, L)
                            cur[i, sl] = cur[i, sl] * wv

                    pltpu.sync_copy(cur, acc.at[dst_v.at[kk]], add=True)

                    @pl.when(kk + DEPTH < KQ)
                    def _():
                        pltpu.async_copy(h_hbm.at[src_v.at[kk + DEPTH]],
                                         cur, csem)

    @pl.when(c == 0)
    def _():
        pipeline(s * KC0, KC0 // KQ)

    @pl.when(c == 1)
    def _():
        pipeline(NS * KC0 + s * KC1, KC1 // KQ)

    plsc.subcore_barrier()
    pltpu.sync_copy(acc.at[pl.ds(base, RPS)],
                    out_hbm.at[c, pl.ds(base, RPS)])


def _sc_scatter(h, src3, dst3, w3):
    mesh = plsc.VectorSubcoreMesh(core_axis_name="c", subcore_axis_name="s")
    cp = pltpu.CompilerParams()
    if "needs_layout_passes" in pltpu.CompilerParams.__dataclass_fields__:
        cp = dataclasses.replace(cp, needs_layout_passes=False)
    fn = pl.kernel(
        _sc_body,
        mesh=mesh,
        out_type=jax.ShapeDtypeStruct((NC, N_PAD, OUT_F), _f32),
        scratch_types=[
            pltpu.VMEM_SHARED((N_PAD, OUT_F), _f32),
            pltpu.VMEM((KQ, CB), jnp.int32),
            pltpu.VMEM((KQ, CB), jnp.int32),
            pltpu.VMEM((KQ, CB), _f32),
        ] + [pltpu.VMEM((CB, OUT_F), _f32) for _ in range(DEPTH)]
          + [pltpu.SemaphoreType.DMA for _ in range(DEPTH)],
        compiler_params=cp,
    )
    return fn(h, src3, dst3, w3)


def _combine_body(acc_ref, b_ref, o_ref):
    o_ref[...] = acc_ref[0] + acc_ref[1] + b_ref[...]


def _combine(acc, b):
    grid = 10
    blk = N_NODES // grid
    return pl.pallas_call(
        _combine_body,
        grid=(grid,),
        in_specs=[
            pl.BlockSpec((NC, blk, OUT_F), lambda i: (0, i, 0)),
            pl.BlockSpec((1, OUT_F), lambda i: (0, 0)),
        ],
        out_specs=pl.BlockSpec((blk, OUT_F), lambda i: (i, 0)),
        out_shape=jax.ShapeDtypeStruct((N_NODES, OUT_F), _f32),
    )(acc, b.reshape(1, OUT_F))


def kernel(x, edge_index, edge_weight, W, b):
    src = edge_index[1].astype(jnp.int32)
    dst = edge_index[0].astype(jnp.int32)
    pad = E_PAD - N_EDGES
    src3 = jnp.pad(src, (0, pad)).reshape(TOT_CHUNKS, CB)
    dst3 = jnp.pad(dst, (0, pad)).reshape(TOT_CHUNKS, CB)
    w3 = jnp.pad(edge_weight.astype(_f32), (0, pad)).reshape(TOT_CHUNKS, CB)

    h = _matmul(x.astype(_f32), W.astype(_f32))
    acc = _sc_scatter(h, src3, dst3, w3)
    return _combine(acc, b.astype(_f32))
